# Initial kernel scaffold; baseline (speedup 1.0000x reference)
import jax
import jax.numpy as jnp
from jax.experimental import pallas as pl


def kernel(f_atoms, f_bonds, edge_index, mol_ids, W_i, W_h, W_o, W_c1, b_c1, W_c2, b_c2, W_c3, b_c3, W_out, b_out):
    raise NotImplementedError("write your pallas kernel here")


# speedup vs baseline: 1.2555x; 1.2555x over previous
import functools

import jax
import jax.numpy as jnp
from jax import lax
from jax.experimental import pallas as pl
from jax.experimental.pallas import tpu as pltpu
from jax.experimental.pallas import tpu_sc as plsc

E = 160000
NATOMS = 10000
NMOLS = 500
H = 384
HS = 128
DIN = 144
DATOM = 128
CLS = 200
BM = 2000
BN = 2000

NS = 16
EBA = 80
EBB = 40
NBLK_A = (E // NS) // EBA
NBLK_B = (E // 2 // NS) // EBB
ZB = 80
NZB = NATOMS // ZB
NMOLS_PAD = 512
M_STRIPE = NMOLS_PAD // NS

_f32 = jnp.float32


@functools.lru_cache(maxsize=None)
def _mesh():
    return plsc.VectorSubcoreMesh(core_axis_name="c", subcore_axis_name="s",
                                  num_cores=2, num_subcores=NS)


def _k0_body(fb, w, out):
    out[...] = jnp.dot(fb[...], w[...], preferred_element_type=_f32)


def _bond_matmul(f_bonds, wi):
    return pl.pallas_call(
        _k0_body,
        grid=(E // BM,),
        in_specs=[
            pl.BlockSpec((BM, DIN), lambda i: (i, 0)),
            pl.BlockSpec((DIN, H), lambda i: (0, 0)),
        ],
        out_specs=pl.BlockSpec((BM, H), lambda i: (i, 0)),
        out_shape=jax.ShapeDtypeStruct((E, H), _f32),
    )(f_bonds, wi)


def _p1_body(x, w, out):
    out[...] = jnp.dot(jnp.maximum(x[...], 0.0), w[...],
                       preferred_element_type=_f32)


def _edge_matmul_relu(inp, wh):
    return pl.pallas_call(
        _p1_body,
        grid=(E // BM,),
        in_specs=[
            pl.BlockSpec((BM, H), lambda i: (i, 0)),
            pl.BlockSpec((H, H), lambda i: (0, 0)),
        ],
        out_specs=pl.BlockSpec((BM, H), lambda i: (i, 0)),
        out_shape=jax.ShapeDtypeStruct((E, H), _f32),
    )(inp, wh)


def _p2_body(ma, mb, w, out):
    out[...] = (jnp.dot(ma[...], w[0:2 * HS, :], preferred_element_type=_f32)
                + jnp.dot(mb[...], w[2 * HS:, :], preferred_element_type=_f32))


def _edge_matmul(ma, mb, wh):
    return pl.pallas_call(
        _p2_body,
        grid=(E // BM,),
        in_specs=[
            pl.BlockSpec((BM, 2 * HS), lambda i: (i, 0)),
            pl.BlockSpec((BM, HS), lambda i: (i, 0)),
            pl.BlockSpec((H, H), lambda i: (0, 0)),
        ],
        out_specs=pl.BlockSpec((BM, H), lambda i: (i, 0)),
        out_shape=jax.ShapeDtypeStruct((E, H), _f32),
    )(ma, mb, wh)


def _q_body(a01, a2p, w, q0, q1, q2):
    q = (jnp.dot(a01[0], w[0:HS, :], preferred_element_type=_f32)
         + jnp.dot(a01[1], w[HS:2 * HS, :], preferred_element_type=_f32)
         + jnp.dot(a2p[0] + a2p[1], w[2 * HS:, :], preferred_element_type=_f32))
    q0[...] = q[:, 0:HS]
    q1[...] = q[:, HS:2 * HS]
    q2[...] = q[:, 2 * HS:]


def _atom_matmul(a01, a2p, wh):
    return pl.pallas_call(
        _q_body,
        grid=(NATOMS // BN,),
        in_specs=[
            pl.BlockSpec((2, BN, HS), lambda i: (0, i, 0)),
            pl.BlockSpec((2, BN, HS), lambda i: (0, i, 0)),
            pl.BlockSpec((H, H), lambda i: (0, 0)),
        ],
        out_specs=(
            pl.BlockSpec((BN, HS), lambda i: (i, 0)),
            pl.BlockSpec((BN, HS), lambda i: (i, 0)),
            pl.BlockSpec((BN, HS), lambda i: (i, 0)),
        ),
        out_shape=(
            jax.ShapeDtypeStruct((NATOMS, HS), _f32),
            jax.ShapeDtypeStruct((NATOMS, HS), _f32),
            jax.ShapeDtypeStruct((NATOMS, HS), _f32),
        ),
    )(a01, a2p, wh)


def _kf_body(a01, a2p, fa, woa, wom, h):
    h[...] = jnp.maximum(
        jnp.dot(fa[...], woa[...], preferred_element_type=_f32)
        + jnp.dot(a01[0], wom[0:HS, :], preferred_element_type=_f32)
        + jnp.dot(a01[1], wom[HS:2 * HS, :], preferred_element_type=_f32)
        + jnp.dot(a2p[0] + a2p[1], wom[2 * HS:, :],
                  preferred_element_type=_f32), 0.0)


def _atom_out(a01, a2p, f_atoms, woa, wom):
    return pl.pallas_call(
        _kf_body,
        grid=(NATOMS // BN,),
        in_specs=[
            pl.BlockSpec((2, BN, HS), lambda i: (0, i, 0)),
            pl.BlockSpec((2, BN, HS), lambda i: (0, i, 0)),
            pl.BlockSpec((BN, DATOM), lambda i: (i, 0)),
            pl.BlockSpec((DATOM, H), lambda i: (0, 0)),
            pl.BlockSpec((H, H), lambda i: (0, 0)),
        ],
        out_specs=pl.BlockSpec((BN, H), lambda i: (i, 0)),
        out_shape=jax.ShapeDtypeStruct((NATOMS, H), _f32),
    )(a01, a2p, f_atoms, woa, wom)


def _cls_body(s, cnt, wc1, b1, wc2, b2, wc3, b3, wout, bout, out):
    sv = s[0] + s[1]
    cv = cnt[0, :, 0:1] + cnt[1, :, 0:1]
    mv = sv[:NMOLS] / jnp.maximum(cv[:NMOLS], 1.0)
    h = jnp.maximum(jnp.dot(mv, wc1[...], preferred_element_type=_f32)
                    + b1[...], 0.0)
    h = jnp.maximum(jnp.dot(h, wc2[...], preferred_element_type=_f32)
                    + b2[...], 0.0)
    h = jnp.maximum(jnp.dot(h, wc3[...], preferred_element_type=_f32)
                    + b3[...], 0.0)
    out[...] = jnp.dot(h, wout[...], preferred_element_type=_f32) + bout[...]


def _classifier(msum, mcnt, wc1, b1, wc2, b2, wc3, b3, wout, bout):
    return pl.pallas_call(
        _cls_body,
        out_shape=jax.ShapeDtypeStruct((NMOLS, 1), _f32),
    )(msum, mcnt, wc1, b1, wc2, b2, wc3, b3, wout, bout)


def _vmem_zero(buf, rows, width):
    def zf(r, carry):
        for k in range(width // 16):
            buf[r, pl.ds(k * 16, 16)] = jnp.zeros((16,), _f32)
        return carry

    lax.fori_loop(0, rows, zf, 0)


def _zero_acc(buf, sh_ref, s):
    _vmem_zero(buf, ZB, HS)
    for t in range((NZB + NS - 1) // NS):
        blk = s + NS * t

        @pl.when(blk < NZB)
        def _():
            pltpu.sync_copy(buf, sh_ref.at[pl.ds(blk * ZB, ZB), :])


def _dump_acc(buf, sh_ref, out_ref, c, s):
    for t in range((NZB + NS - 1) // NS):
        blk = s + NS * t

        @pl.when(blk < NZB)
        def _():
            pltpu.sync_copy(sh_ref.at[pl.ds(blk * ZB, ZB), :], buf)
            pltpu.sync_copy(buf, out_ref.at[c, pl.ds(blk * ZB, ZB), :])


def _col(c):
    return pl.multiple_of(c * HS, HS)


def _seg_a_body(inp_ref, dst_ref, a_ref, b_x, b_z, b_i, sh_a, sem):
    c = lax.axis_index("c")
    s = lax.axis_index("s")
    _zero_acc(b_z, sh_a, s)
    plsc.subcore_barrier()
    col = _col(c)

    def blk(t, carry):
        e0 = s * (E // NS) + t * EBA
        pltpu.sync_copy(inp_ref.at[pl.ds(e0, EBA), pl.ds(col, HS)], b_x)

        def row(r, carry2):
            for k in range(HS // 16):
                d = pl.ds(k * 16, 16)
                b_x[r, d] = jnp.maximum(b_x[r, d], 0.0)
            return carry2

        lax.fori_loop(0, EBA, row, 0)
        pltpu.sync_copy(dst_ref.at[pl.ds(e0, EBA)], b_i)
        pltpu.sync_copy(b_x, sh_a.at[b_i], add=True)
        return carry

    lax.fori_loop(0, NBLK_A, blk, 0)
    plsc.subcore_barrier()
    _dump_acc(b_z, sh_a, a_ref, c, s)


def _seg_b_body(inp_ref, dst_ref, a_ref, b_x, b_z, b_i, sh_a, sem):
    c = lax.axis_index("c")
    s = lax.axis_index("s")
    _zero_acc(b_z, sh_a, s)
    plsc.subcore_barrier()

    def blk(t, carry):
        e0 = c * (E // 2) + s * (E // 2 // NS) + t * EBB
        pltpu.sync_copy(inp_ref.at[pl.ds(e0, EBB), pl.ds(2 * HS, HS)], b_x)

        def row(r, carry2):
            for k in range(HS // 16):
                d = pl.ds(k * 16, 16)
                b_x[r, d] = jnp.maximum(b_x[r, d], 0.0)
            return carry2

        lax.fori_loop(0, EBB, row, 0)
        pltpu.sync_copy(dst_ref.at[pl.ds(e0, EBB)], b_i)
        pltpu.sync_copy(b_x, sh_a.at[b_i], add=True)
        return carry

    lax.fori_loop(0, NBLK_B, blk, 0)
    plsc.subcore_barrier()
    _dump_acc(b_z, sh_a, a_ref, c, s)


def _segsum_relu(inp, dst, body, eb):
    k = pl.kernel(
        body,
        out_type=jax.ShapeDtypeStruct((2, NATOMS, HS), _f32),
        mesh=_mesh(),
        scratch_types=[
            pltpu.VMEM((eb, HS), _f32),
            pltpu.VMEM((ZB, HS), _f32),
            pltpu.VMEM((eb,), jnp.int32),
            pltpu.VMEM_SHARED((NATOMS, HS), _f32),
            pltpu.SemaphoreType.DMA,
        ],
    )
    return k(inp, dst)


def _comb_compute(b_a, b_p, b_q, eb):
    def pair(pr, carry2):
        r0 = pr * 2
        r1 = r0 + 1
        for k in range(HS // 16):
            d = pl.ds(k * 16, 16)
            z0 = b_a[r0, d] + b_q[r0, d] - b_p[r1, d]
            z1 = b_a[r1, d] + b_q[r1, d] - b_p[r0, d]
            b_q[r0, d] = jnp.maximum(z0, 0.0)
            b_q[r1, d] = jnp.maximum(z1, 0.0)
        return carry2

    lax.fori_loop(0, eb // 2, pair, 0)


def _comb_a_body(inp_ref, p_ref, q0_ref, q1_ref, src_ref, dst_ref,
                 a_ref, m_ref, b_a, b_p, b_q, b_z, b_is, b_id, sh_a, sem):
    c = lax.axis_index("c")
    s = lax.axis_index("s")
    _zero_acc(b_z, sh_a, s)
    plsc.subcore_barrier()
    col = _col(c)

    def blk(t, carry):
        e0 = s * (E // NS) + t * EBA
        pltpu.sync_copy(src_ref.at[pl.ds(e0, EBA)], b_is)

        @pl.when(c == 0)
        def _():
            pltpu.async_copy(q0_ref.at[b_is], b_q, sem).wait()

        @pl.when(c == 1)
        def _():
            pltpu.async_copy(q1_ref.at[b_is], b_q, sem).wait()

        pltpu.sync_copy(inp_ref.at[pl.ds(e0, EBA), pl.ds(col, HS)], b_a)
        pltpu.sync_copy(p_ref.at[pl.ds(e0, EBA), pl.ds(col, HS)], b_p)
        _comb_compute(b_a, b_p, b_q, EBA)
        pltpu.sync_copy(dst_ref.at[pl.ds(e0, EBA)], b_id)
        pltpu.sync_copy(b_q, sh_a.at[b_id], add=True)
        pltpu.sync_copy(b_q, m_ref.at[pl.ds(e0, EBA), pl.ds(col, HS)])
        return carry

    lax.fori_loop(0, NBLK_A, blk, 0)
    plsc.subcore_barrier()
    _dump_acc(b_z, sh_a, a_ref, c, s)


def _comb_a_nom_body(inp_ref, p_ref, q0_ref, q1_ref, src_ref, dst_ref,
                     a_ref, b_a, b_p, b_q, b_z, b_is, b_id, sh_a, sem):
    c = lax.axis_index("c")
    s = lax.axis_index("s")
    _zero_acc(b_z, sh_a, s)
    plsc.subcore_barrier()
    col = _col(c)

    def blk(t, carry):
        e0 = s * (E // NS) + t * EBA
        pltpu.sync_copy(src_ref.at[pl.ds(e0, EBA)], b_is)

        @pl.when(c == 0)
        def _():
            pltpu.async_copy(q0_ref.at[b_is], b_q, sem).wait()

        @pl.when(c == 1)
        def _():
            pltpu.async_copy(q1_ref.at[b_is], b_q, sem).wait()

        pltpu.sync_copy(inp_ref.at[pl.ds(e0, EBA), pl.ds(col, HS)], b_a)
        pltpu.sync_copy(p_ref.at[pl.ds(e0, EBA), pl.ds(col, HS)], b_p)
        _comb_compute(b_a, b_p, b_q, EBA)
        pltpu.sync_copy(dst_ref.at[pl.ds(e0, EBA)], b_id)
        pltpu.sync_copy(b_q, sh_a.at[b_id], add=True)
        return carry

    lax.fori_loop(0, NBLK_A, blk, 0)
    plsc.subcore_barrier()
    _dump_acc(b_z, sh_a, a_ref, c, s)


def _comb_b_body(inp_ref, p_ref, q2_ref, src_ref, dst_ref,
                 a_ref, m_ref, b_a, b_p, b_q, b_z, b_is, b_id, sh_a, sem,
                 *, write_m):
    c = lax.axis_index("c")
    s = lax.axis_index("s")
    _zero_acc(b_z, sh_a, s)
    plsc.subcore_barrier()

    def blk(t, carry):
        e0 = c * (E // 2) + s * (E // 2 // NS) + t * EBB
        pltpu.sync_copy(src_ref.at[pl.ds(e0, EBB)], b_is)
        pltpu.async_copy(q2_ref.at[b_is], b_q, sem).wait()
        pltpu.sync_copy(inp_ref.at[pl.ds(e0, EBB), pl.ds(2 * HS, HS)], b_a)
        pltpu.sync_copy(p_ref.at[pl.ds(e0, EBB), pl.ds(2 * HS, HS)], b_p)
        _comb_compute(b_a, b_p, b_q, EBB)
        pltpu.sync_copy(dst_ref.at[pl.ds(e0, EBB)], b_id)
        pltpu.sync_copy(b_q, sh_a.at[b_id], add=True)
        if write_m:
            pltpu.sync_copy(b_q, m_ref.at[pl.ds(e0, EBB), :])
        return carry

    lax.fori_loop(0, NBLK_B, blk, 0)
    plsc.subcore_barrier()
    _dump_acc(b_z, sh_a, a_ref, c, s)


def _combine_a(inp, p, q0, q1, src, dst, write_m):
    if write_m:
        out_type = (jax.ShapeDtypeStruct((2, NATOMS, HS), _f32),
                    jax.ShapeDtypeStruct((E, 2 * HS), _f32))
        body = _comb_a_body
    else:
        out_type = jax.ShapeDtypeStruct((2, NATOMS, HS), _f32)
        body = _comb_a_nom_body
    k = pl.kernel(
        body,
        out_type=out_type,
        mesh=_mesh(),
        scratch_types=[
            pltpu.VMEM((EBA, HS), _f32),
            pltpu.VMEM((EBA, HS), _f32),
            pltpu.VMEM((EBA, HS), _f32),
            pltpu.VMEM((ZB, HS), _f32),
            pltpu.VMEM((EBA,), jnp.int32),
            pltpu.VMEM((EBA,), jnp.int32),
            pltpu.VMEM_SHARED((NATOMS, HS), _f32),
            pltpu.SemaphoreType.DMA,
        ],
    )
    return k(inp, p, q0, q1, src, dst)


def _combine_b(inp, p, q2, src, dst, write_m):
    out_type = (jax.ShapeDtypeStruct((2, NATOMS, HS), _f32),
                jax.ShapeDtypeStruct((E, HS), _f32))
    k = pl.kernel(
        functools.partial(_comb_b_body, write_m=write_m),
        out_type=out_type,
        mesh=_mesh(),
        scratch_types=[
            pltpu.VMEM((EBB, HS), _f32),
            pltpu.VMEM((EBB, HS), _f32),
            pltpu.VMEM((EBB, HS), _f32),
            pltpu.VMEM((ZB, HS), _f32),
            pltpu.VMEM((EBB,), jnp.int32),
            pltpu.VMEM((EBB,), jnp.int32),
            pltpu.VMEM_SHARED((NATOMS, HS), _f32),
            pltpu.SemaphoreType.DMA,
        ],
    )
    return k(inp, p, q2, src, dst)


def _mol_body(h_ref, mol_ref, s_out, c_out,
              b_x0, b_x1, b_x2, b_one, b_i, sh_s0, sh_s1, sh_s2, sh_c, sem):
    c = lax.axis_index("c")
    s = lax.axis_index("s")
    stripe = pl.ds(s * M_STRIPE, M_STRIPE)
    _vmem_zero(b_x0, M_STRIPE, HS)
    pltpu.sync_copy(b_x0.at[pl.ds(0, M_STRIPE), :], sh_s0.at[stripe, :])
    pltpu.sync_copy(b_x0.at[pl.ds(0, M_STRIPE), :], sh_s1.at[stripe, :])
    pltpu.sync_copy(b_x0.at[pl.ds(0, M_STRIPE), :], sh_s2.at[stripe, :])
    pltpu.sync_copy(b_x0.at[pl.ds(0, M_STRIPE), :], sh_c.at[stripe, :])

    def of(r, carry):
        for k in range(HS // 16):
            b_one[r, pl.ds(k * 16, 16)] = jnp.full((16,), 1.0, _f32)
        return carry

    lax.fori_loop(0, EBB, of, 0)
    plsc.subcore_barrier()

    nblk = NATOMS // 2 // EBB
    for t in range((nblk + NS - 1) // NS):
        blk = s + NS * t

        @pl.when(blk < nblk)
        def _():
            a0 = c * (NATOMS // 2) + blk * EBB
            pltpu.sync_copy(h_ref.at[pl.ds(a0, EBB), pl.ds(0, HS)], b_x0)
            pltpu.sync_copy(h_ref.at[pl.ds(a0, EBB), pl.ds(HS, HS)], b_x1)
            pltpu.sync_copy(h_ref.at[pl.ds(a0, EBB), pl.ds(2 * HS, HS)], b_x2)
            pltpu.sync_copy(mol_ref.at[pl.ds(a0, EBB)], b_i)
            pltpu.sync_copy(b_x0, sh_s0.at[b_i], add=True)
            pltpu.sync_copy(b_x1, sh_s1.at[b_i], add=True)
            pltpu.sync_copy(b_x2, sh_s2.at[b_i], add=True)
            pltpu.sync_copy(b_one, sh_c.at[b_i], add=True)

    plsc.subcore_barrier()
    pltpu.sync_copy(sh_s0.at[stripe, :], s_out.at[c, stripe, pl.ds(0, HS)])
    pltpu.sync_copy(sh_s1.at[stripe, :], s_out.at[c, stripe, pl.ds(HS, HS)])
    pltpu.sync_copy(sh_s2.at[stripe, :], s_out.at[c, stripe, pl.ds(2 * HS, HS)])
    pltpu.sync_copy(sh_c.at[stripe, :], c_out.at[c, stripe, :])


def _mol_mean_sums(h, mol_ids):
    k = pl.kernel(
        _mol_body,
        out_type=(jax.ShapeDtypeStruct((2, NMOLS_PAD, H), _f32),
                  jax.ShapeDtypeStruct((2, NMOLS_PAD, HS), _f32)),
        mesh=_mesh(),
        scratch_types=[
            pltpu.VMEM((EBB, HS), _f32),
            pltpu.VMEM((EBB, HS), _f32),
            pltpu.VMEM((EBB, HS), _f32),
            pltpu.VMEM((EBB, HS), _f32),
            pltpu.VMEM((EBB,), jnp.int32),
            pltpu.VMEM_SHARED((NMOLS_PAD, HS), _f32),
            pltpu.VMEM_SHARED((NMOLS_PAD, HS), _f32),
            pltpu.VMEM_SHARED((NMOLS_PAD, HS), _f32),
            pltpu.VMEM_SHARED((NMOLS_PAD, HS), _f32),
            pltpu.SemaphoreType.DMA,
        ],
    )
    return k(h, mol_ids)


def kernel(f_atoms, f_bonds, edge_index, mol_ids, W_i, W_h, W_o,
           W_c1, b_c1, W_c2, b_c2, W_c3, b_c3, W_out, b_out):
    src = edge_index[0].astype(jnp.int32)
    dst = edge_index[1].astype(jnp.int32)
    mol = mol_ids.astype(jnp.int32)

    hid = W_i.shape[1]
    pad = H - hid

    wi = jnp.pad(W_i, ((0, 0), (0, pad)))
    wh = jnp.pad(W_h, ((0, pad), (0, pad)))
    woa = jnp.pad(W_o[:DATOM], ((0, 0), (0, pad)))
    wom = jnp.pad(W_o[DATOM:], ((0, pad), (0, pad)))
    wc1 = jnp.pad(W_c1, ((0, pad), (0, 0)))

    inp = _bond_matmul(f_bonds, wi)

    a1_01 = _segsum_relu(inp, dst, _seg_a_body, EBA)
    a1_2p = _segsum_relu(inp, dst, _seg_b_body, EBB)

    p1 = _edge_matmul_relu(inp, wh)
    q1_0, q1_1, q1_2 = _atom_matmul(a1_01, a1_2p, wh)

    a2_01, m2a = _combine_a(inp, p1, q1_0, q1_1, src, dst, write_m=True)
    a2_2p, m2b = _combine_b(inp, p1, q1_2, src, dst, write_m=True)

    p2 = _edge_matmul(m2a, m2b, wh)
    q2_0, q2_1, q2_2 = _atom_matmul(a2_01, a2_2p, wh)

    a3_01 = _combine_a(inp, p2, q2_0, q2_1, src, dst, write_m=False)
    a3_2p, _ = _combine_b(inp, p2, q2_2, src, dst, write_m=False)

    h = _atom_out(a3_01, a3_2p, f_atoms, woa, wom)

    msum, mcnt = _mol_mean_sums(h, mol)

    logits = _classifier(
        msum, mcnt,
        wc1, b_c1.reshape(1, CLS),
        W_c2, b_c2.reshape(1, CLS),
        W_c3, b_c3.reshape(1, CLS),
        W_out, b_out.reshape(1, 1),
    )
    return logits
